# Initial kernel scaffold; baseline (speedup 1.0000x reference)
import jax
import jax.numpy as jnp
from jax.experimental import pallas as pl


def kernel(boxes, scores):
    raise NotImplementedError("write your pallas kernel here")


# speedup vs baseline: 6.4627x; 6.4627x over previous
import functools

import jax
import jax.numpy as jnp
from jax import lax
from jax.experimental import pallas as pl
from jax.experimental.pallas import tpu as pltpu
from jax.experimental.pallas import tpu_sc as plsc

N = 20000
P = 20480
NT = 16
SHARD = P // NT
VPT = SHARD // 16
IOU_THRES = 0.45
CONF_THRES = 0.25
MAX_DET = 300
NEG = -1.0


def _nms_body(x1_hbm, y1_hbm, x2_hbm, y2_hbm, act_hbm, out_hbm,
              lx1, ly1, lx2, ly2, lact, pub, recv, outb, recs_sh):
    t = lax.axis_index("s")
    base = t * SHARD
    lane = lax.iota(jnp.int32, 16)
    zeros16i = jnp.zeros((16,), jnp.int32)
    zeros16f = jnp.zeros((16,), jnp.float32)

    pltpu.sync_copy(x1_hbm.at[pl.ds(base, SHARD)], lx1)
    pltpu.sync_copy(y1_hbm.at[pl.ds(base, SHARD)], ly1)
    pltpu.sync_copy(x2_hbm.at[pl.ds(base, SHARD)], lx2)
    pltpu.sync_copy(y2_hbm.at[pl.ds(base, SHARD)], ly2)
    pltpu.sync_copy(act_hbm.at[pl.ds(base, SHARD)], lact)

    @pl.when(t == 0)
    def _zero_out():
        def zbody(j, _):
            outb[pl.ds(j * 16, 16)] = zeros16f
            return 0
        lax.fori_loop(0, MAX_DET, zbody, 0)

    def round_body(i, cont):
        def amax_body(j, carry):
            vmax, vidx = carry
            idxv = j * 16 + lane
            v = plsc.load_gather(lact, [idxv])
            m = v > vmax
            return jnp.where(m, v, vmax), jnp.where(m, idxv, vidx)

        vmax, vidx = lax.fori_loop(
            0, VPT, amax_body,
            (jnp.full((16,), -2.0, jnp.float32), zeros16i))
        gmax = jnp.max(vmax)
        sidx = jnp.min(jnp.where(vmax == gmax, vidx, jnp.int32(1 << 30)))
        sidx_v = jnp.full((16,), sidx, jnp.int32)

        wx1 = plsc.load_gather(lx1, [sidx_v])
        wy1 = plsc.load_gather(ly1, [sidx_v])
        wx2 = plsc.load_gather(lx2, [sidx_v])
        wy2 = plsc.load_gather(ly2, [sidx_v])
        gidx_f = (base + sidx).astype(jnp.float32)
        rec = jnp.where(lane == 0, jnp.full((16,), gmax, jnp.float32),
              jnp.where(lane == 1, jnp.full((16,), gidx_f, jnp.float32),
              jnp.where(lane == 2, wx1,
              jnp.where(lane == 3, wy1,
              jnp.where(lane == 4, wx2,
              jnp.where(lane == 5, wy2, zeros16f))))))
        pub[...] = rec
        pltpu.sync_copy(pub, recs_sh.at[pl.ds(t * 16, 16)])
        plsc.subcore_barrier()
        pltpu.sync_copy(recs_sh, recv)
        plsc.subcore_barrier()

        s_scores = plsc.load_gather(recv, [lane * 16])
        gbest = jnp.max(s_scores)
        wt = jnp.min(jnp.where(s_scores == gbest, lane, jnp.int32(9999)))
        wt_v = jnp.full((16,), wt, jnp.int32)

        def fld(k):
            return plsc.load_gather(recv, [wt_v * 16 + k])

        wgidx_v = fld(1).astype(jnp.int32)
        wx1v, wy1v, wx2v, wy2v = fld(2), fld(3), fld(4), fld(5)
        keep = gbest > CONF_THRES

        @pl.when(keep)
        def _select():
            warea = (wx2v - wx1v) * (wy2v - wy1v)

            def sup_body(j, _):
                idxv = j * 16 + lane
                x1v = plsc.load_gather(lx1, [idxv])
                y1v = plsc.load_gather(ly1, [idxv])
                x2v = plsc.load_gather(lx2, [idxv])
                y2v = plsc.load_gather(ly2, [idxv])
                av = (x2v - x1v) * (y2v - y1v)
                xx1 = jnp.maximum(x1v, wx1v)
                yy1 = jnp.maximum(y1v, wy1v)
                xx2 = jnp.minimum(x2v, wx2v)
                yy2 = jnp.minimum(y2v, wy2v)
                inter = (jnp.maximum(xx2 - xx1, 0.0)
                         * jnp.maximum(yy2 - yy1, 0.0))
                denom = av + warea - inter + 1e-9
                kill = (inter > IOU_THRES * denom) | (base + idxv == wgidx_v)
                actv = plsc.load_gather(lact, [idxv])
                plsc.store_scatter(lact, [idxv],
                                   jnp.where(kill, NEG, actv))
                return 0

            lax.fori_loop(0, VPT, sup_body, 0)

            @pl.when(t == 0)
            def _store_row():
                row = jnp.where(lane == 0, wx1v,
                      jnp.where(lane == 1, wy1v,
                      jnp.where(lane == 2, wx2v,
                      jnp.where(lane == 3, wy2v,
                      jnp.where(lane == 4,
                                jnp.full((16,), gbest, jnp.float32),
                                zeros16f)))))
                outb[pl.ds(i * 16, 16)] = row

        return keep.astype(jnp.int32)

    def outer_body(i, cont):
        return lax.cond(cont == 1, lambda: round_body(i, cont),
                        lambda: jnp.int32(0))

    lax.fori_loop(0, MAX_DET, outer_body, jnp.int32(1))

    @pl.when(t == 0)
    def _flush():
        pltpu.sync_copy(outb, out_hbm)


@functools.partial(
    pl.kernel,
    out_type=jax.ShapeDtypeStruct((MAX_DET * 16,), jnp.float32),
    mesh=plsc.VectorSubcoreMesh(core_axis_name="c", subcore_axis_name="s",
                                num_cores=1, num_subcores=16),
    compiler_params=pltpu.CompilerParams(needs_layout_passes=False),
    scratch_types=[
        pltpu.VMEM((SHARD,), jnp.float32),
        pltpu.VMEM((SHARD,), jnp.float32),
        pltpu.VMEM((SHARD,), jnp.float32),
        pltpu.VMEM((SHARD,), jnp.float32),
        pltpu.VMEM((SHARD,), jnp.float32),
        pltpu.VMEM((16,), jnp.float32),
        pltpu.VMEM((NT * 16,), jnp.float32),
        pltpu.VMEM((MAX_DET * 16,), jnp.float32),
        pltpu.VMEM_SHARED((NT * 16,), jnp.float32),
    ],
)
def _nms_sc(x1_hbm, y1_hbm, x2_hbm, y2_hbm, act_hbm, out_hbm,
            lx1, ly1, lx2, ly2, lact, pub, recv, outb, recs_sh):
    _nms_body(x1_hbm, y1_hbm, x2_hbm, y2_hbm, act_hbm, out_hbm,
              lx1, ly1, lx2, ly2, lact, pub, recv, outb, recs_sh)


def kernel(boxes, scores):
    pad = P - N
    x1 = jnp.pad(boxes[:, 0], (0, pad))
    y1 = jnp.pad(boxes[:, 1], (0, pad))
    x2 = jnp.pad(boxes[:, 2], (0, pad))
    y2 = jnp.pad(boxes[:, 3], (0, pad))
    act = jnp.pad(scores, (0, pad), constant_values=NEG)
    flat = _nms_sc(x1, y1, x2, y2, act)
    return flat.reshape(MAX_DET, 16)[:, :5]


# speedup vs baseline: 8.4619x; 1.3094x over previous
import functools

import jax
import jax.numpy as jnp
from jax import lax
from jax.experimental import pallas as pl
from jax.experimental.pallas import tpu as pltpu
from jax.experimental.pallas import tpu_sc as plsc

N = 20000
P = 20480
NT = 16
SHARD = P // NT
VPT = SHARD // 16
REC = 16
IOU_THRES = 0.45
CONF_THRES = 0.25
MAX_DET = 300
NEG = -1.0


def _nms_body(x1_hbm, y1_hbm, x2_hbm, y2_hbm, act_hbm, out_hbm,
              lx1, ly1, lx2, ly2, lact, pub, recv, outb, recs_sh):
    t = lax.axis_index("s")
    base = t * SHARD
    lane = lax.iota(jnp.int32, 16)
    zeros16f = jnp.zeros((16,), jnp.float32)

    pltpu.sync_copy(x1_hbm.at[pl.ds(base, SHARD)], lx1)
    pltpu.sync_copy(y1_hbm.at[pl.ds(base, SHARD)], ly1)
    pltpu.sync_copy(x2_hbm.at[pl.ds(base, SHARD)], lx2)
    pltpu.sync_copy(y2_hbm.at[pl.ds(base, SHARD)], ly2)
    pltpu.sync_copy(act_hbm.at[pl.ds(base, SHARD)], lact)

    @pl.when(t == 0)
    def _zero_out():
        def zbody(j, _):
            outb[pl.ds(j * 16, 16)] = zeros16f
            return 0
        lax.fori_loop(0, MAX_DET, zbody, 0)

    def round_body(k, carry):
        def do_round():
            _, pwx1, pwy1, pwx2, pwy2, pwgidx = carry
            pwarea = (pwx2 - pwx1) * (pwy2 - pwy1)

            def pass_body(j, c2):
                vmax, vidx = c2
                o = j * 16
                lidx = o + lane
                x1v = lx1[pl.ds(o, 16)]
                y1v = ly1[pl.ds(o, 16)]
                x2v = lx2[pl.ds(o, 16)]
                y2v = ly2[pl.ds(o, 16)]
                actv = lact[pl.ds(o, 16)]
                av = (x2v - x1v) * (y2v - y1v)
                xx1 = jnp.maximum(x1v, pwx1)
                yy1 = jnp.maximum(y1v, pwy1)
                xx2 = jnp.minimum(x2v, pwx2)
                yy2 = jnp.minimum(y2v, pwy2)
                inter = (jnp.maximum(xx2 - xx1, 0.0)
                         * jnp.maximum(yy2 - yy1, 0.0))
                denom = av + pwarea - inter + 1e-9
                kill = (inter > IOU_THRES * denom) | (base + lidx == pwgidx)
                newact = jnp.where(kill, NEG, actv)
                lact[pl.ds(o, 16)] = newact
                m = newact > vmax
                return (jnp.where(m, newact, vmax),
                        jnp.where(m, lidx, vidx))

            vmax, vidx = lax.fori_loop(
                0, VPT, pass_body,
                (jnp.full((16,), -2.0, jnp.float32),
                 jnp.zeros((16,), jnp.int32)))
            gmax = jnp.max(vmax)
            sidx = jnp.min(jnp.where(vmax == gmax, vidx, jnp.int32(1 << 30)))
            sidx_v = jnp.full((16,), sidx, jnp.int32)

            wx1 = plsc.load_gather(lx1, [sidx_v])
            wy1 = plsc.load_gather(ly1, [sidx_v])
            wx2 = plsc.load_gather(lx2, [sidx_v])
            wy2 = plsc.load_gather(ly2, [sidx_v])
            gidx_f = (base + sidx).astype(jnp.float32)
            rec = jnp.where(lane == 0, jnp.full((16,), gmax, jnp.float32),
                  jnp.where(lane == 1, jnp.full((16,), gidx_f, jnp.float32),
                  jnp.where(lane == 2, wx1,
                  jnp.where(lane == 3, wy1,
                  jnp.where(lane == 4, wx2,
                  jnp.where(lane == 5, wy2, zeros16f))))))
            pub[...] = rec
            par = (k % 2) * (NT * REC)
            pltpu.sync_copy(pub, recs_sh.at[pl.ds(par + t * REC, REC)])
            plsc.subcore_barrier()
            pltpu.sync_copy(recs_sh.at[pl.ds(par, NT * REC)], recv)

            s_scores = plsc.load_gather(recv, [lane * REC])
            gbest = jnp.max(s_scores)
            wt = jnp.min(jnp.where(s_scores == gbest, lane, jnp.int32(9999)))
            wt_v = jnp.full((16,), wt * REC, jnp.int32)

            def fld(c):
                return plsc.load_gather(recv, [wt_v + c])

            wgidx_v = fld(1).astype(jnp.int32)
            wx1v, wy1v, wx2v, wy2v = fld(2), fld(3), fld(4), fld(5)
            keep = gbest > CONF_THRES

            @pl.when(keep & (t == 0))
            def _store_row():
                row = jnp.where(lane == 0, wx1v,
                      jnp.where(lane == 1, wy1v,
                      jnp.where(lane == 2, wx2v,
                      jnp.where(lane == 3, wy2v,
                      jnp.where(lane == 4,
                                jnp.full((16,), gbest, jnp.float32),
                                zeros16f)))))
                outb[pl.ds(k * 16, 16)] = row

            return (keep.astype(jnp.int32),
                    wx1v, wy1v, wx2v, wy2v, wgidx_v)

        def skip():
            return (jnp.int32(0),) + tuple(carry[1:])

        return lax.cond(carry[0] == 1, do_round, skip)

    init = (jnp.int32(1), zeros16f, zeros16f, zeros16f, zeros16f,
            jnp.full((16,), -1, jnp.int32))
    lax.fori_loop(0, MAX_DET, round_body, init)

    @pl.when(t == 0)
    def _flush():
        pltpu.sync_copy(outb, out_hbm)


@functools.partial(
    pl.kernel,
    out_type=jax.ShapeDtypeStruct((MAX_DET * 16,), jnp.float32),
    mesh=plsc.VectorSubcoreMesh(core_axis_name="c", subcore_axis_name="s",
                                num_cores=1, num_subcores=16),
    compiler_params=pltpu.CompilerParams(needs_layout_passes=False),
    scratch_types=[
        pltpu.VMEM((SHARD,), jnp.float32),
        pltpu.VMEM((SHARD,), jnp.float32),
        pltpu.VMEM((SHARD,), jnp.float32),
        pltpu.VMEM((SHARD,), jnp.float32),
        pltpu.VMEM((SHARD,), jnp.float32),
        pltpu.VMEM((16,), jnp.float32),
        pltpu.VMEM((NT * REC,), jnp.float32),
        pltpu.VMEM((MAX_DET * 16,), jnp.float32),
        pltpu.VMEM_SHARED((2 * NT * REC,), jnp.float32),
    ],
)
def _nms_sc(x1_hbm, y1_hbm, x2_hbm, y2_hbm, act_hbm, out_hbm,
            lx1, ly1, lx2, ly2, lact, pub, recv, outb, recs_sh):
    _nms_body(x1_hbm, y1_hbm, x2_hbm, y2_hbm, act_hbm, out_hbm,
              lx1, ly1, lx2, ly2, lact, pub, recv, outb, recs_sh)


def kernel(boxes, scores):
    pad = P - N
    x1 = jnp.pad(boxes[:, 0], (0, pad))
    y1 = jnp.pad(boxes[:, 1], (0, pad))
    x2 = jnp.pad(boxes[:, 2], (0, pad))
    y2 = jnp.pad(boxes[:, 3], (0, pad))
    act = jnp.pad(scores, (0, pad), constant_values=NEG)
    flat = _nms_sc(x1, y1, x2, y2, act)
    return flat.reshape(MAX_DET, 16)[:, :5]


# speedup vs baseline: 24.6844x; 2.9171x over previous
import functools

import jax
import jax.numpy as jnp
from jax import lax
from jax.experimental import pallas as pl
from jax.experimental.pallas import tpu as pltpu
from jax.experimental.pallas import tpu_sc as plsc

N = 20000
P = 20480
NT = 16
SHARD = P // NT
VPT = SHARD // 16
CAP = 128
ACAP = SHARD + 16
AGRP = ACAP // 16
REC = 16
IOU_THRES = 0.45
CONF_THRES = 0.25
MAX_DET = 300
NEG = -1.0
BIG = 1 << 30


def _nms_body(x1_hbm, y1_hbm, x2_hbm, y2_hbm, sc_hbm, out_hbm,
              lx1, ly1, lx2, ly2, lsc,
              ax1, ay1, ax2, ay2, aact, agidx,
              pub, recv, keptb, recs_sh):
    t = lax.axis_index("s")
    base = t * SHARD
    lane = lax.iota(jnp.int32, 16)
    zeros16f = jnp.zeros((16,), jnp.float32)
    neg16f = jnp.full((16,), NEG, jnp.float32)

    pltpu.sync_copy(x1_hbm.at[pl.ds(base, SHARD)], lx1)
    pltpu.sync_copy(y1_hbm.at[pl.ds(base, SHARD)], ly1)
    pltpu.sync_copy(x2_hbm.at[pl.ds(base, SHARD)], lx2)
    pltpu.sync_copy(y2_hbm.at[pl.ds(base, SHARD)], ly2)
    pltpu.sync_copy(sc_hbm.at[pl.ds(base, SHARD)], lsc)

    def zbody(j, _):
        keptb[pl.ds(j * 16, 16)] = zeros16f
        return 0
    lax.fori_loop(0, MAX_DET, zbody, 0)

    def mc_body(j, c):
        vm, vc = c
        s = lsc[pl.ds(j * 16, 16)]
        return jnp.maximum(vm, s), vc + (s > CONF_THRES).astype(jnp.float32)

    vm, vc = lax.fori_loop(0, VPT, mc_body,
                           (jnp.full((16,), -2.0, jnp.float32), zeros16f))
    maxsc = jnp.max(vm)
    cnt_conf = jnp.sum(vc)

    def bs_body(it, c):
        lo, hi = c
        mid = (lo + hi) * 0.5

        def cb(j, a):
            s = lsc[pl.ds(j * 16, 16)]
            return a + (s > mid).astype(jnp.float32)

        cnt = jnp.sum(lax.fori_loop(0, VPT, cb, zeros16f))
        big = cnt > float(CAP)
        return jnp.where(big, mid, lo), jnp.where(big, hi, mid)

    _, hi = lax.fori_loop(0, 16, bs_body,
                          (jnp.float32(CONF_THRES), maxsc + 1.0))
    T = jnp.where(cnt_conf <= float(CAP), jnp.float32(CONF_THRES), hi)

    def prefill(j, _):
        o = j * 16
        aact[pl.ds(o, 16)] = neg16f
        agidx[pl.ds(o, 16)] = jnp.full((16,), -7, jnp.int32)
        return 0

    def compact(thresh):
        lax.fori_loop(0, AGRP, prefill, 0)

        def cp(j, off):
            o = j * 16
            s = lsc[pl.ds(o, 16)]
            mask = s > thresh
            mi = mask.astype(jnp.int32)
            cs = plsc.cumsum(mi)
            pos = off + cs - mi
            plsc.store_scatter(aact, [pos], s, mask=mask)
            plsc.store_scatter(ax1, [pos], lx1[pl.ds(o, 16)], mask=mask)
            plsc.store_scatter(ay1, [pos], ly1[pl.ds(o, 16)], mask=mask)
            plsc.store_scatter(ax2, [pos], lx2[pl.ds(o, 16)], mask=mask)
            plsc.store_scatter(ay2, [pos], ly2[pl.ds(o, 16)], mask=mask)
            plsc.store_scatter(agidx, [pos], base + o + lane, mask=mask)
            return off + jnp.max(cs)

        return lax.fori_loop(0, VPT, cp, jnp.int32(0))

    cnt0 = compact(T)
    ng0 = (cnt0 + 15) // 16
    more = cnt_conf > cnt0.astype(jnp.float32)

    def active_argmax(ngroups):
        def am(g, c):
            vmx, vix = c
            a = aact[pl.ds(g * 16, 16)]
            m = a > vmx
            return jnp.where(m, a, vmx), jnp.where(m, g * 16 + lane, vix)

        vmx, vix = lax.fori_loop(0, ngroups, am,
                                 (jnp.full((16,), -2.0, jnp.float32),
                                  jnp.zeros((16,), jnp.int32)))
        gm = jnp.max(vmx)
        sp = jnp.min(jnp.where(vmx == gm, vix, BIG))
        return gm, sp

    def round_body(k, carry):
        def do_round():
            (_, pwsc, pwx1, pwy1, pwx2, pwy2, pwgidx, ng, refilled) = carry

            @pl.when(k > 0)
            def _kst():
                prow = jnp.where(lane == 0, pwx1,
                       jnp.where(lane == 1, pwy1,
                       jnp.where(lane == 2, pwx2,
                       jnp.where(lane == 3, pwy2,
                       jnp.where(lane == 4, pwsc, zeros16f)))))
                keptb[pl.ds((k - 1) * 16, 16)] = prow

            pwarea = (pwx2 - pwx1) * (pwy2 - pwy1)

            def pass_body(j, c2):
                vmx, vix = c2
                o = j * 16
                x1v = ax1[pl.ds(o, 16)]
                y1v = ay1[pl.ds(o, 16)]
                x2v = ax2[pl.ds(o, 16)]
                y2v = ay2[pl.ds(o, 16)]
                actv = aact[pl.ds(o, 16)]
                gidxv = agidx[pl.ds(o, 16)]
                av = (x2v - x1v) * (y2v - y1v)
                xx1 = jnp.maximum(x1v, pwx1)
                yy1 = jnp.maximum(y1v, pwy1)
                xx2 = jnp.minimum(x2v, pwx2)
                yy2 = jnp.minimum(y2v, pwy2)
                inter = (jnp.maximum(xx2 - xx1, 0.0)
                         * jnp.maximum(yy2 - yy1, 0.0))
                denom = av + pwarea - inter + 1e-9
                kill = (inter > IOU_THRES * denom) | (gidxv == pwgidx)
                newact = jnp.where(kill, NEG, actv)
                aact[pl.ds(o, 16)] = newact
                m = newact > vmx
                return (jnp.where(m, newact, vmx),
                        jnp.where(m, o + lane, vix))

            vmx, vix = lax.fori_loop(
                0, ng, pass_body,
                (jnp.full((16,), -2.0, jnp.float32),
                 jnp.zeros((16,), jnp.int32)))
            gmax = jnp.max(vmx)
            spos = jnp.min(jnp.where(vmx == gmax, vix, BIG))

            def refill():
                cnt = compact(jnp.float32(CONF_THRES))
                ngr = (cnt + 15) // 16

                def kf(j, _):
                    def fld(c):
                        return plsc.load_gather(
                            keptb, [jnp.full((16,), j * 16 + c, jnp.int32)])

                    kx1, ky1, kx2, ky2 = fld(0), fld(1), fld(2), fld(3)
                    ka = (kx2 - kx1) * (ky2 - ky1)

                    def kg(g, _2):
                        o = g * 16
                        x1v = ax1[pl.ds(o, 16)]
                        y1v = ay1[pl.ds(o, 16)]
                        x2v = ax2[pl.ds(o, 16)]
                        y2v = ay2[pl.ds(o, 16)]
                        actv = aact[pl.ds(o, 16)]
                        av = (x2v - x1v) * (y2v - y1v)
                        xx1 = jnp.maximum(x1v, kx1)
                        yy1 = jnp.maximum(y1v, ky1)
                        xx2 = jnp.minimum(x2v, kx2)
                        yy2 = jnp.minimum(y2v, ky2)
                        inter = (jnp.maximum(xx2 - xx1, 0.0)
                                 * jnp.maximum(yy2 - yy1, 0.0))
                        denom = av + ka - inter + 1e-9
                        kill = inter > IOU_THRES * denom
                        aact[pl.ds(o, 16)] = jnp.where(kill, NEG, actv)
                        return 0

                    lax.fori_loop(0, ngr, kg, 0)
                    return 0

                lax.fori_loop(0, k, kf, 0)
                gm2, sp2 = active_argmax(ngr)
                return gm2, sp2, ngr, jnp.int32(1)

            need = ((gmax < -0.5) & more & (refilled == 0))
            gmax, spos, ng, refilled = lax.cond(
                need, refill, lambda: (gmax, spos, ng, refilled))

            spos_v = jnp.full((16,), spos, jnp.int32)
            wx1 = plsc.load_gather(ax1, [spos_v])
            wy1 = plsc.load_gather(ay1, [spos_v])
            wx2 = plsc.load_gather(ax2, [spos_v])
            wy2 = plsc.load_gather(ay2, [spos_v])
            gidx_f = plsc.load_gather(agidx, [spos_v]).astype(jnp.float32)
            rec = jnp.where(lane == 0, jnp.full((16,), gmax, jnp.float32),
                  jnp.where(lane == 1, gidx_f,
                  jnp.where(lane == 2, wx1,
                  jnp.where(lane == 3, wy1,
                  jnp.where(lane == 4, wx2,
                  jnp.where(lane == 5, wy2, zeros16f))))))
            pub[...] = rec
            par = (k % 2) * (NT * REC)
            pltpu.sync_copy(pub, recs_sh.at[pl.ds(par + t * REC, REC)])
            plsc.subcore_barrier()
            pltpu.sync_copy(recs_sh.at[pl.ds(par, NT * REC)], recv)

            s_scores = plsc.load_gather(recv, [lane * REC])
            gbest = jnp.max(s_scores)
            wt = jnp.min(jnp.where(s_scores == gbest, lane, jnp.int32(9999)))
            wt_v = jnp.full((16,), wt * REC, jnp.int32)

            def rfld(c):
                return plsc.load_gather(recv, [wt_v + c])

            wgidx_v = rfld(1).astype(jnp.int32)
            wx1v, wy1v, wx2v, wy2v = rfld(2), rfld(3), rfld(4), rfld(5)
            keep = gbest > CONF_THRES

            return (keep.astype(jnp.int32),
                    jnp.full((16,), gbest, jnp.float32),
                    wx1v, wy1v, wx2v, wy2v, wgidx_v, ng, refilled)

        def skip():
            return (jnp.int32(0),) + tuple(carry[1:])

        return lax.cond(carry[0] == 1, do_round, skip)

    init = (jnp.int32(1), zeros16f, zeros16f, zeros16f, zeros16f, zeros16f,
            jnp.full((16,), -1, jnp.int32), ng0, jnp.int32(0))
    fin = lax.fori_loop(0, MAX_DET, round_body, init)

    @pl.when(fin[0] == 1)
    def _last():
        frow = jnp.where(lane == 0, fin[2],
               jnp.where(lane == 1, fin[3],
               jnp.where(lane == 2, fin[4],
               jnp.where(lane == 3, fin[5],
               jnp.where(lane == 4, fin[1], zeros16f)))))
        keptb[pl.ds((MAX_DET - 1) * 16, 16)] = frow

    @pl.when(t == 0)
    def _flush():
        pltpu.sync_copy(keptb, out_hbm)


@functools.partial(
    pl.kernel,
    out_type=jax.ShapeDtypeStruct((MAX_DET * 16,), jnp.float32),
    mesh=plsc.VectorSubcoreMesh(core_axis_name="c", subcore_axis_name="s",
                                num_cores=1, num_subcores=16),
    compiler_params=pltpu.CompilerParams(needs_layout_passes=False),
    scratch_types=[
        pltpu.VMEM((SHARD,), jnp.float32),
        pltpu.VMEM((SHARD,), jnp.float32),
        pltpu.VMEM((SHARD,), jnp.float32),
        pltpu.VMEM((SHARD,), jnp.float32),
        pltpu.VMEM((SHARD,), jnp.float32),
        pltpu.VMEM((ACAP,), jnp.float32),
        pltpu.VMEM((ACAP,), jnp.float32),
        pltpu.VMEM((ACAP,), jnp.float32),
        pltpu.VMEM((ACAP,), jnp.float32),
        pltpu.VMEM((ACAP,), jnp.float32),
        pltpu.VMEM((ACAP,), jnp.int32),
        pltpu.VMEM((16,), jnp.float32),
        pltpu.VMEM((NT * REC,), jnp.float32),
        pltpu.VMEM((MAX_DET * 16,), jnp.float32),
        pltpu.VMEM_SHARED((2 * NT * REC,), jnp.float32),
    ],
)
def _nms_sc(x1_hbm, y1_hbm, x2_hbm, y2_hbm, sc_hbm, out_hbm,
            lx1, ly1, lx2, ly2, lsc,
            ax1, ay1, ax2, ay2, aact, agidx,
            pub, recv, keptb, recs_sh):
    _nms_body(x1_hbm, y1_hbm, x2_hbm, y2_hbm, sc_hbm, out_hbm,
              lx1, ly1, lx2, ly2, lsc,
              ax1, ay1, ax2, ay2, aact, agidx,
              pub, recv, keptb, recs_sh)


def kernel(boxes, scores):
    pad = P - N
    x1 = jnp.pad(boxes[:, 0], (0, pad))
    y1 = jnp.pad(boxes[:, 1], (0, pad))
    x2 = jnp.pad(boxes[:, 2], (0, pad))
    y2 = jnp.pad(boxes[:, 3], (0, pad))
    sc = jnp.pad(scores, (0, pad), constant_values=NEG)
    flat = _nms_sc(x1, y1, x2, y2, sc)
    return flat.reshape(MAX_DET, 16)[:, :5]
